# initial kernel scaffold (unmeasured)
import jax
import jax.numpy as jnp
from jax import lax
from jax.experimental import pallas as pl
from jax.experimental.pallas import tpu as pltpu


def kernel(
    x,
):
    def body(*refs):
        pass

    out_shape = jax.ShapeDtypeStruct(..., jnp.float32)
    return pl.pallas_call(body, out_shape=out_shape)(...)


# baseline (device time: 8825 ns/iter reference)
import jax
import jax.numpy as jnp
from jax import lax
from jax.experimental import pallas as pl
from jax.experimental.pallas import tpu as pltpu

K = 8
NEG = float(jnp.finfo(jnp.float32).min)


def _topk_desc(v, k):
    cols = []
    for _ in range(k):
        mx = jnp.max(v, axis=1, keepdims=True)
        cols.append(mx)
        v = jnp.where(v == mx, NEG, v)
    return jnp.concatenate(cols, axis=1)


def kernel(x):
    m, n = x.shape

    def body(x_ref, out_ref, cand_ref, peer_ref, send_sem, recv_sem):
        my_x = lax.axis_index("x")
        my_y = lax.axis_index("y")
        my_z = lax.axis_index("z")
        peer = (my_x, my_y, 1 - my_z)

        barrier_sem = pltpu.get_barrier_semaphore()
        pl.semaphore_signal(
            barrier_sem,
            inc=1,
            device_id=peer,
            device_id_type=pl.DeviceIdType.MESH,
        )
        pl.semaphore_wait(barrier_sem, 1)

        mine = _topk_desc(x_ref[:, :], K)
        cand_ref[:, :] = mine

        rdma = pltpu.make_async_remote_copy(
            src_ref=cand_ref,
            dst_ref=peer_ref,
            send_sem=send_sem,
            recv_sem=recv_sem,
            device_id=peer,
            device_id_type=pl.DeviceIdType.MESH,
        )
        rdma.start()
        rdma.wait()

        both = jnp.concatenate([mine, peer_ref[:, :]], axis=1)
        out_ref[:, :] = _topk_desc(both, K)

    return pl.pallas_call(
        body,
        out_shape=jax.ShapeDtypeStruct((m, K), jnp.float32),
        in_specs=[pl.BlockSpec(memory_space=pltpu.VMEM)],
        out_specs=pl.BlockSpec(memory_space=pltpu.VMEM),
        scratch_shapes=[
            pltpu.VMEM((m, K), jnp.float32),
            pltpu.VMEM((m, K), jnp.float32),
            pltpu.SemaphoreType.DMA,
            pltpu.SemaphoreType.DMA,
        ],
        compiler_params=pltpu.CompilerParams(collective_id=0),
    )(x)
